# baseline (speedup 1.0000x reference)
import functools
import math

import jax
import jax.numpy as jnp
import numpy as np
from jax import lax
from jax.experimental import pallas as pl
from jax.experimental.pallas import tpu as pltpu
from jax.experimental.pallas import tpu_sc as plsc

_N = 10000
_E = 320000
_H = 128
_NRBF = 64
_NB = 64
_CUTOFF = 5.0

_NC = 2
_NS = 16
_NW = _NC * _NS

_E_PAD = 327680
_EW = _E_PAD // _NW
_KE = 128
_EBLK = _EW // _KE

_CB0 = 52
_CB1 = 108

_N_PAD = 10240
_KN = 40
_NBLK = (_N_PAD // _NW) // _KN

_STRIPE = _N_PAD // _NS

_TN = 2000
_TE = 1024

_OFFS = np.linspace(0.0, _CUTOFF, _NRBF).astype(np.float32)
_COEFF = float(-0.5 / (_OFFS[1] - _OFFS[0]) ** 2)

_NBUF = 4
_NBUF_SC = 2


def _sc_mesh():
    return plsc.VectorSubcoreMesh(core_axis_name="c", subcore_axis_name="s")


_SC_PARAMS = pltpu.CompilerParams(needs_layout_passes=False)


def _sc_edge_d2(posx, posy, posz, src, dst):
    @functools.partial(
        pl.kernel,
        mesh=_sc_mesh(),
        compiler_params=_SC_PARAMS,
        out_type=jax.ShapeDtypeStruct((_E_PAD,), jnp.float32),
        scratch_types=[
            pltpu.VMEM((_N_PAD,), jnp.float32),
            pltpu.VMEM((_N_PAD,), jnp.float32),
            pltpu.VMEM((_N_PAD,), jnp.float32),
            pltpu.VMEM((_EW,), jnp.int32),
            pltpu.VMEM((_EW,), jnp.int32),
            pltpu.VMEM((_EW,), jnp.float32),
        ],
    )
    def k(px_h, py_h, pz_h, src_h, dst_h, out_h, px, py, pz, sb, db, d2b):
        cid = lax.axis_index("c")
        sid = lax.axis_index("s")
        base = (cid * _NS + sid) * _EW
        pltpu.sync_copy(px_h, px)
        pltpu.sync_copy(py_h, py)
        pltpu.sync_copy(pz_h, pz)
        pltpu.sync_copy(src_h.at[pl.ds(base, _EW)], sb)
        pltpu.sync_copy(dst_h.at[pl.ds(base, _EW)], db)

        def body(j, carry):
            o = j * 16
            si = sb[pl.ds(o, 16)]
            di = db[pl.ds(o, 16)]
            dx = plsc.load_gather(px, [di]) - plsc.load_gather(px, [si])
            dy = plsc.load_gather(py, [di]) - plsc.load_gather(py, [si])
            dz = plsc.load_gather(pz, [di]) - plsc.load_gather(pz, [si])
            d2b[pl.ds(o, 16)] = dx * dx + dy * dy + dz * dz
            return carry

        lax.fori_loop(0, _EW // 16, body, 0)
        pltpu.sync_copy(d2b, out_h.at[pl.ds(base, _EW)])

    return k(posx, posy, posz, src, dst)


def _sc_gather_rows(table, idx, nblocks, k_rows):
    total = _NW * nblocks * k_rows
    per_w = nblocks * k_rows
    nsteps = nblocks // _NBUF

    @functools.partial(
        pl.kernel,
        mesh=_sc_mesh(),
        compiler_params=_SC_PARAMS,
        out_type=jax.ShapeDtypeStruct((total, _H), jnp.float32),
        scratch_types=[
            pltpu.VMEM((per_w,), jnp.int32),
            pltpu.VMEM((_NBUF, k_rows, _H), jnp.float32),
        ] + [pltpu.SemaphoreType.DMA] * (2 * _NBUF),
    )
    def k(table_h, idx_h, out_h, idx_v, bufs, *sems):
        gsem = sems[:_NBUF]
        ssem = sems[_NBUF:]
        cid = lax.axis_index("c")
        sid = lax.axis_index("s")
        wid = cid * _NS + sid
        base = wid * per_w
        pltpu.sync_copy(idx_h.at[pl.ds(base, per_w)], idx_v)

        def fire(blk, s):
            pltpu.async_copy(
                table_h.at[idx_v.at[pl.ds(blk * k_rows, k_rows)]],
                bufs.at[s], gsem[s])

        for s in range(_NBUF):
            fire(s, s)

        def body(kk, carry):
            for s in range(_NBUF):
                blk = kk * _NBUF + s
                pltpu.make_async_copy(
                    table_h.at[idx_v.at[pl.ds(0, k_rows)]],
                    bufs.at[s], gsem[s]).wait()
                pltpu.async_copy(
                    bufs.at[s],
                    out_h.at[pl.ds(base + blk * k_rows, k_rows)], ssem[s])
                pltpu.make_async_copy(
                    bufs.at[s],
                    out_h.at[pl.ds(base, k_rows)], ssem[s]).wait()
                nxt = blk + _NBUF

                @pl.when(nxt < nblocks)
                def _():
                    fire(nxt, s)
            return carry

        lax.fori_loop(0, nsteps, body, 0)

    return k(table, idx)


_KG = 64
_GBLK = _EW // _KG
_YSTRIPE = _N // _NS


def _sc_conv_msg(table, wf, idx):
    @functools.partial(
        pl.kernel,
        mesh=_sc_mesh(),
        compiler_params=_SC_PARAMS,
        out_type=jax.ShapeDtypeStruct((_E_PAD, _H), jnp.float32),
        scratch_types=[
            pltpu.VMEM((_EW,), jnp.int32),
            pltpu.VMEM((2, _KG, _H), jnp.float32),
            pltpu.VMEM((2, _KG, _H), jnp.float32),
            pltpu.VMEM_SHARED((_N, _H), jnp.float32),
        ] + [pltpu.SemaphoreType.DMA] * 6,
    )
    def k(table_h, wf_h, idx_h, out_h, idx_v, bufs, wfb, ytab, *sems):
        gsem = sems[0:2]
        wsem = sems[2:4]
        ssem = sems[4:6]
        cid = lax.axis_index("c")
        sid = lax.axis_index("s")
        wid = cid * _NS + sid
        base = wid * _EW

        @pl.when(sid < 15)
        def _():
            pltpu.sync_copy(table_h.at[pl.ds(sid * 640, 640)],
                            ytab.at[pl.ds(sid * 640, 640)])

        @pl.when(sid == 15)
        def _():
            pltpu.sync_copy(table_h.at[pl.ds(9600, _N - 9600)],
                            ytab.at[pl.ds(9600, _N - 9600)])

        pltpu.sync_copy(idx_h.at[pl.ds(base, _EW)], idx_v)
        plsc.subcore_barrier()

        def fire(blk, s):
            pltpu.async_copy(
                ytab.at[idx_v.at[pl.ds(blk * _KG, _KG)]],
                bufs.at[s], gsem[s])
            pltpu.async_copy(
                wf_h.at[pl.ds(base + blk * _KG, _KG)], wfb.at[s], wsem[s])

        for s in range(2):
            fire(s, s)

        def body(kk, carry):
            for s in range(2):
                blk = kk * 2 + s
                pltpu.make_async_copy(
                    table_h.at[pl.ds(0, _KG)], bufs.at[s], gsem[s]).wait()
                pltpu.make_async_copy(
                    wf_h.at[pl.ds(0, _KG)], wfb.at[s], wsem[s]).wait()

                def mrow(r, carry2):
                    for h in range(8):
                        sl = pl.ds(h * 16, 16)
                        bufs[s, r, sl] = bufs[s, r, sl] * wfb[s, r, sl]
                    return carry2

                lax.fori_loop(0, _KG, mrow, 0)
                pltpu.async_copy(
                    bufs.at[s],
                    out_h.at[pl.ds(base + blk * _KG, _KG)], ssem[s])
                pltpu.make_async_copy(
                    bufs.at[s],
                    out_h.at[pl.ds(0, _KG)], ssem[s]).wait()
                nxt = blk + 2

                @pl.when(nxt < _GBLK)
                def _():
                    fire(nxt, s)
            return carry

        lax.fori_loop(0, _GBLK // 2, body, 0)

    return k(table, wf, idx)


def _sc_scatter_add(msg, dst, zeros_pad):
    nsteps = _EBLK // _NBUF_SC

    @functools.partial(
        pl.kernel,
        mesh=_sc_mesh(),
        compiler_params=_SC_PARAMS,
        out_type=jax.ShapeDtypeStruct((2 * _N_PAD, _H), jnp.float32),
        scratch_types=[
            pltpu.VMEM((_EBLK, _KE), jnp.int32),
            pltpu.VMEM((_NBUF_SC, _KE, _H), jnp.float32),
            pltpu.VMEM_SHARED((_N_PAD, _H), jnp.float32),
        ] + [pltpu.SemaphoreType.DMA] * (2 * _NBUF_SC),
    )
    def k(msg_h, dst_h, z_h, out_h, idx_v, bufs, agg, *sems):
        lsem = sems[:_NBUF_SC]
        csem = sems[_NBUF_SC:]
        cid = lax.axis_index("c")
        sid = lax.axis_index("s")
        wid = cid * _NS + sid
        pltpu.sync_copy(z_h.at[pl.ds(sid * _STRIPE, _STRIPE)],
                        agg.at[pl.ds(sid * _STRIPE, _STRIPE)])
        pltpu.sync_copy(dst_h.at[pl.ds(wid * _EBLK, _EBLK)], idx_v)
        plsc.subcore_barrier()

        def fire(blk, s):
            pltpu.async_copy(
                msg_h.at[pl.ds((wid * _EBLK + blk) * _KE, _KE)],
                bufs.at[s], lsem[s])

        for s in range(_NBUF_SC):
            fire(s, s)

        def body(kk, carry):
            for s in range(_NBUF_SC):
                blk = kk * _NBUF_SC + s
                pltpu.make_async_copy(
                    msg_h.at[pl.ds(0, _KE)], bufs.at[s], lsem[s]).wait()
                pltpu.async_copy(bufs.at[s], agg.at[idx_v.at[blk]],
                                 csem[s], add=True)
                pltpu.make_async_copy(
                    bufs.at[s], agg.at[idx_v.at[0]], csem[s]).wait()
                nxt = blk + _NBUF_SC

                @pl.when(nxt < _EBLK)
                def _():
                    fire(nxt, s)
            return carry

        lax.fori_loop(0, nsteps, body, 0)
        plsc.subcore_barrier()
        pltpu.sync_copy(agg.at[pl.ds(sid * _STRIPE, _STRIPE)],
                        out_h.at[pl.ds(cid * _N_PAD + sid * _STRIPE, _STRIPE)])

    return k(msg, dst, zeros_pad)


def _tc_node_linear(x, w):
    def body(x_ref, w_ref, o_ref):
        o_ref[...] = jnp.dot(x_ref[...], w_ref[...],
                             preferred_element_type=jnp.float32)

    return pl.pallas_call(
        body,
        grid=(_N // _TN,),
        in_specs=[pl.BlockSpec((_TN, _H), lambda i: (i, 0)),
                  pl.BlockSpec((_H, _H), lambda i: (0, 0))],
        out_specs=pl.BlockSpec((_TN, _H), lambda i: (i, 0)),
        out_shape=jax.ShapeDtypeStruct((_N, _H), jnp.float32),
    )(x, w)


def _tc_filter_all(d2, w1, b1, w2, b2):
    step = _CUTOFF / (_NRBF - 1)

    def body(d2_ref, w1_ref, b1_ref, w2_ref, b2_ref, o_ref):
        offs = lax.broadcasted_iota(jnp.int32, (1, _NRBF), 1).astype(
            jnp.float32) * step
        d = jnp.sqrt(d2_ref[...] + 1e-12)
        rbf = jnp.exp(_COEFF * (d - offs) ** 2)
        t = jnp.tanh(jnp.dot(rbf, w1_ref[0],
                             preferred_element_type=jnp.float32) + b1_ref[0])
        wf = jnp.dot(t, w2_ref[0],
                     preferred_element_type=jnp.float32) + b2_ref[0]
        cc = 0.5 * (jnp.cos(d * (math.pi / _CUTOFF)) + 1.0)
        cc = cc * (d < _CUTOFF).astype(jnp.float32)
        o_ref[0] = wf * cc

    return pl.pallas_call(
        body,
        grid=(3, _E_PAD // _TE),
        in_specs=[pl.BlockSpec((_TE, 1), lambda l, i: (i, 0)),
                  pl.BlockSpec((1, _NRBF, _H), lambda l, i: (l, 0, 0)),
                  pl.BlockSpec((1, 1, _H), lambda l, i: (l, 0, 0)),
                  pl.BlockSpec((1, _H, _H), lambda l, i: (l, 0, 0)),
                  pl.BlockSpec((1, 1, _H), lambda l, i: (l, 0, 0))],
        out_specs=pl.BlockSpec((1, _TE, _H), lambda l, i: (l, i, 0)),
        out_shape=jax.ShapeDtypeStruct((3, _E_PAD, _H), jnp.float32),
    )(d2, w1, b1, w2, b2)


def _tc_update(agg_a, agg_b, x, w_out, b_out, w_lin, b_lin):
    def body(a_ref, b_ref, x_ref, wo_ref, bo_ref, wl_ref, bl_ref, o_ref):
        agg = a_ref[...] + b_ref[...]
        h = jnp.tanh(jnp.dot(agg, wo_ref[...],
                             preferred_element_type=jnp.float32) + bo_ref[...])
        h = jnp.dot(h, wl_ref[...],
                    preferred_element_type=jnp.float32) + bl_ref[...]
        o_ref[...] = x_ref[...] + h

    return pl.pallas_call(
        body,
        grid=(_N // _TN,),
        in_specs=[pl.BlockSpec((_TN, _H), lambda i: (i, 0)),
                  pl.BlockSpec((_TN, _H), lambda i: (i, 0)),
                  pl.BlockSpec((_TN, _H), lambda i: (i, 0)),
                  pl.BlockSpec((_H, _H), lambda i: (0, 0)),
                  pl.BlockSpec((1, _H), lambda i: (0, 0)),
                  pl.BlockSpec((_H, _H), lambda i: (0, 0)),
                  pl.BlockSpec((1, _H), lambda i: (0, 0))],
        out_specs=pl.BlockSpec((_TN, _H), lambda i: (i, 0)),
        out_shape=jax.ShapeDtypeStruct((_N, _H), jnp.float32),
    )(agg_a, agg_b, x, w_out, b_out, w_lin, b_lin)


def _tc_update_ylin(agg_a, agg_b, x, w_out, b_out, w_lin, b_lin, w_next):
    def body(a_ref, b_ref, x_ref, wo_ref, bo_ref, wl_ref, bl_ref, wn_ref,
             o_ref, y_ref):
        agg = a_ref[...] + b_ref[...]
        h = jnp.tanh(jnp.dot(agg, wo_ref[...],
                             preferred_element_type=jnp.float32) + bo_ref[...])
        h = jnp.dot(h, wl_ref[...],
                    preferred_element_type=jnp.float32) + bl_ref[...]
        xn = x_ref[...] + h
        o_ref[...] = xn
        y_ref[...] = jnp.dot(xn, wn_ref[...],
                             preferred_element_type=jnp.float32)

    return pl.pallas_call(
        body,
        grid=(_N // _TN,),
        in_specs=[pl.BlockSpec((_TN, _H), lambda i: (i, 0)),
                  pl.BlockSpec((_TN, _H), lambda i: (i, 0)),
                  pl.BlockSpec((_TN, _H), lambda i: (i, 0)),
                  pl.BlockSpec((_H, _H), lambda i: (0, 0)),
                  pl.BlockSpec((1, _H), lambda i: (0, 0)),
                  pl.BlockSpec((_H, _H), lambda i: (0, 0)),
                  pl.BlockSpec((1, _H), lambda i: (0, 0)),
                  pl.BlockSpec((_H, _H), lambda i: (0, 0))],
        out_specs=[pl.BlockSpec((_TN, _H), lambda i: (i, 0)),
                   pl.BlockSpec((_TN, _H), lambda i: (i, 0))],
        out_shape=[jax.ShapeDtypeStruct((_N, _H), jnp.float32),
                   jax.ShapeDtypeStruct((_N, _H), jnp.float32)],
    )(agg_a, agg_b, x, w_out, b_out, w_lin, b_lin, w_next)


def _tc_readout(x, seg, w1, b1, w2, b2):
    def body(x_ref, s_ref, w1_ref, b1_ref, w2_ref, b2_ref, o_ref):
        t = jnp.tanh(jnp.dot(x_ref[...], w1_ref[...],
                             preferred_element_type=jnp.float32) + b1_ref[...])
        e = jnp.dot(t, w2_ref[...],
                    preferred_element_type=jnp.float32) + b2_ref[...]
        cols = lax.broadcasted_iota(jnp.int32, (1, _NB), 1)
        m = (s_ref[...] == cols).astype(jnp.float32)
        part = jnp.sum(e * m, axis=0, keepdims=True)

        @pl.when(pl.program_id(0) == 0)
        def _():
            o_ref[...] = jnp.zeros_like(o_ref)

        o_ref[...] += part

    return pl.pallas_call(
        body,
        grid=(_N // _TN,),
        in_specs=[pl.BlockSpec((_TN, _H), lambda i: (i, 0)),
                  pl.BlockSpec((_TN, 1), lambda i: (i, 0)),
                  pl.BlockSpec((_H, _H // 2), lambda i: (0, 0)),
                  pl.BlockSpec((1, _H // 2), lambda i: (0, 0)),
                  pl.BlockSpec((_H // 2, 1), lambda i: (0, 0)),
                  pl.BlockSpec((1, 1), lambda i: (0, 0))],
        out_specs=pl.BlockSpec((1, _NB), lambda i: (0, 0)),
        out_shape=jax.ShapeDtypeStruct((1, _NB), jnp.float32),
    )(x, seg, w1, b1, w2, b2)


def kernel(pos, emb, filt_W1, filt_b1, filt_W2, filt_b2, lin_in_W, lin_out_W,
           lin_out_b, lin_W, lin_b, out1_W, out1_b, out2_W, out2_b,
           atom_types, edge_index, batch):
    f32 = jnp.float32
    pos = pos.astype(f32)
    src = edge_index[0].astype(jnp.int32)
    dst = edge_index[1].astype(jnp.int32)
    src_p = jnp.pad(src, (0, _E_PAD - _E))
    dst_p = jnp.pad(dst, (0, _E_PAD - _E), constant_values=_N)
    dst_blk = dst_p.reshape(_NW * _EBLK, _KE)
    posx = jnp.pad(pos[:, 0], (0, _N_PAD - _N))
    posy = jnp.pad(pos[:, 1], (0, _N_PAD - _N))
    posz = jnp.pad(pos[:, 2], (0, _N_PAD - _N))
    types_p = jnp.pad(atom_types.astype(jnp.int32), (0, _N_PAD - _N))
    zeros_pad = jnp.zeros((_N_PAD, _H), f32)
    seg = batch.astype(jnp.int32).reshape(_N, 1)

    d2 = _sc_edge_d2(posx, posy, posz, src_p, dst_p).reshape(_E_PAD, 1)
    x = _sc_gather_rows(emb.astype(f32), types_p, _NBLK, _KN)[:_N]
    wf_all = _tc_filter_all(d2,
                            filt_W1.astype(f32),
                            filt_b1.astype(f32).reshape(3, 1, _H),
                            filt_W2.astype(f32),
                            filt_b2.astype(f32).reshape(3, 1, _H))
    y = _tc_node_linear(x, lin_in_W[0].astype(f32))

    for i in range(3):
        msg = _sc_conv_msg(y, wf_all[i], src_p)
        agg2 = _sc_scatter_add(msg, dst_blk, zeros_pad)
        args = (agg2[:_N], agg2[_N_PAD:_N_PAD + _N], x,
                lin_out_W[i].astype(f32),
                lin_out_b[i].astype(f32).reshape(1, _H),
                lin_W[i].astype(f32),
                lin_b[i].astype(f32).reshape(1, _H))
        if i < 2:
            x, y = _tc_update_ylin(*args, lin_in_W[i + 1].astype(f32))
        else:
            x = _tc_update(*args)

    energy = _tc_readout(x, seg,
                         out1_W.astype(f32),
                         out1_b.astype(f32).reshape(1, _H // 2),
                         out2_W.astype(f32),
                         out2_b.astype(f32).reshape(1, 1))
    return energy.reshape(_NB)


# speedup vs baseline: 1.2462x; 1.2462x over previous
import functools
import math

import jax
import jax.numpy as jnp
import numpy as np
from jax import lax
from jax.experimental import pallas as pl
from jax.experimental.pallas import tpu as pltpu
from jax.experimental.pallas import tpu_sc as plsc

_N = 10000
_E = 320000
_H = 128
_NRBF = 64
_NB = 64
_CUTOFF = 5.0

_NC = 2
_NS = 16
_NW = _NC * _NS

_E_PAD = 327680
_EW = _E_PAD // _NW
_KE = 128
_EBLK = _EW // _KE

_CB0 = 52
_CB1 = 108

_N_PAD = 10240
_KN = 40
_NBLK = (_N_PAD // _NW) // _KN

_STRIPE = _N_PAD // _NS

_TN = 2000
_TE = 1024

_OFFS = np.linspace(0.0, _CUTOFF, _NRBF).astype(np.float32)
_COEFF = float(-0.5 / (_OFFS[1] - _OFFS[0]) ** 2)

_NBUF = 4
_NBUF_SC = 2


def _sc_mesh():
    return plsc.VectorSubcoreMesh(core_axis_name="c", subcore_axis_name="s")


_SC_PARAMS = pltpu.CompilerParams(needs_layout_passes=False)


def _sc_edge_d2(posx, posy, posz, src, dst):
    @functools.partial(
        pl.kernel,
        mesh=_sc_mesh(),
        compiler_params=_SC_PARAMS,
        out_type=jax.ShapeDtypeStruct((_E_PAD,), jnp.float32),
        scratch_types=[
            pltpu.VMEM((_N_PAD,), jnp.float32),
            pltpu.VMEM((_N_PAD,), jnp.float32),
            pltpu.VMEM((_N_PAD,), jnp.float32),
            pltpu.VMEM((_EW,), jnp.int32),
            pltpu.VMEM((_EW,), jnp.int32),
            pltpu.VMEM((_EW,), jnp.float32),
        ],
    )
    def k(px_h, py_h, pz_h, src_h, dst_h, out_h, px, py, pz, sb, db, d2b):
        cid = lax.axis_index("c")
        sid = lax.axis_index("s")
        base = (cid * _NS + sid) * _EW
        pltpu.sync_copy(px_h, px)
        pltpu.sync_copy(py_h, py)
        pltpu.sync_copy(pz_h, pz)
        pltpu.sync_copy(src_h.at[pl.ds(base, _EW)], sb)
        pltpu.sync_copy(dst_h.at[pl.ds(base, _EW)], db)

        def body(j, carry):
            o = j * 16
            si = sb[pl.ds(o, 16)]
            di = db[pl.ds(o, 16)]
            dx = plsc.load_gather(px, [di]) - plsc.load_gather(px, [si])
            dy = plsc.load_gather(py, [di]) - plsc.load_gather(py, [si])
            dz = plsc.load_gather(pz, [di]) - plsc.load_gather(pz, [si])
            d2b[pl.ds(o, 16)] = dx * dx + dy * dy + dz * dz
            return carry

        lax.fori_loop(0, _EW // 16, body, 0)
        pltpu.sync_copy(d2b, out_h.at[pl.ds(base, _EW)])

    return k(posx, posy, posz, src, dst)


def _sc_gather_rows(table, idx, nblocks, k_rows):
    total = _NW * nblocks * k_rows
    per_w = nblocks * k_rows
    nsteps = nblocks // _NBUF

    @functools.partial(
        pl.kernel,
        mesh=_sc_mesh(),
        compiler_params=_SC_PARAMS,
        out_type=jax.ShapeDtypeStruct((total, _H), jnp.float32),
        scratch_types=[
            pltpu.VMEM((per_w,), jnp.int32),
            pltpu.VMEM((_NBUF, k_rows, _H), jnp.float32),
        ] + [pltpu.SemaphoreType.DMA] * (2 * _NBUF),
    )
    def k(table_h, idx_h, out_h, idx_v, bufs, *sems):
        gsem = sems[:_NBUF]
        ssem = sems[_NBUF:]
        cid = lax.axis_index("c")
        sid = lax.axis_index("s")
        wid = cid * _NS + sid
        base = wid * per_w
        pltpu.sync_copy(idx_h.at[pl.ds(base, per_w)], idx_v)

        def fire(blk, s):
            pltpu.async_copy(
                table_h.at[idx_v.at[pl.ds(blk * k_rows, k_rows)]],
                bufs.at[s], gsem[s])

        for s in range(_NBUF):
            fire(s, s)

        def body(kk, carry):
            for s in range(_NBUF):
                blk = kk * _NBUF + s
                pltpu.make_async_copy(
                    table_h.at[idx_v.at[pl.ds(0, k_rows)]],
                    bufs.at[s], gsem[s]).wait()
                pltpu.async_copy(
                    bufs.at[s],
                    out_h.at[pl.ds(base + blk * k_rows, k_rows)], ssem[s])
                pltpu.make_async_copy(
                    bufs.at[s],
                    out_h.at[pl.ds(base, k_rows)], ssem[s]).wait()
                nxt = blk + _NBUF

                @pl.when(nxt < nblocks)
                def _():
                    fire(nxt, s)
            return carry

        lax.fori_loop(0, nsteps, body, 0)

    return k(table, idx)


_KG = 64
_GBLK = _EW // _KG
_YSTRIPE = _N // _NS


def _sc_gather_edges(table, idx):
    @functools.partial(
        pl.kernel,
        mesh=_sc_mesh(),
        compiler_params=_SC_PARAMS,
        out_type=jax.ShapeDtypeStruct((_E_PAD, _H), jnp.float32),
        scratch_types=[
            pltpu.VMEM((_EW,), jnp.int32),
            pltpu.VMEM((_NBUF, _KG, _H), jnp.float32),
            pltpu.VMEM_SHARED((_N, _H), jnp.float32),
        ] + [pltpu.SemaphoreType.DMA] * (2 * _NBUF),
    )
    def k(table_h, idx_h, out_h, idx_v, bufs, ytab, *sems):
        gsem = sems[:_NBUF]
        ssem = sems[_NBUF:]
        cid = lax.axis_index("c")
        sid = lax.axis_index("s")
        wid = cid * _NS + sid
        base = wid * _EW

        @pl.when(sid < 15)
        def _():
            pltpu.sync_copy(table_h.at[pl.ds(sid * 640, 640)],
                            ytab.at[pl.ds(sid * 640, 640)])

        @pl.when(sid == 15)
        def _():
            pltpu.sync_copy(table_h.at[pl.ds(9600, _N - 9600)],
                            ytab.at[pl.ds(9600, _N - 9600)])

        pltpu.sync_copy(idx_h.at[pl.ds(base, _EW)], idx_v)
        plsc.subcore_barrier()

        def fire(blk, s):
            pltpu.async_copy(
                ytab.at[idx_v.at[pl.ds(blk * _KG, _KG)]],
                bufs.at[s], gsem[s])

        for s in range(_NBUF):
            fire(s, s)

        def body(kk, carry):
            for s in range(_NBUF):
                blk = kk * _NBUF + s
                pltpu.make_async_copy(
                    table_h.at[pl.ds(0, _KG)],
                    bufs.at[s], gsem[s]).wait()
                pltpu.async_copy(
                    bufs.at[s],
                    out_h.at[pl.ds(base + blk * _KG, _KG)], ssem[s])
                pltpu.make_async_copy(
                    bufs.at[s],
                    out_h.at[pl.ds(0, _KG)], ssem[s]).wait()
                nxt = blk + _NBUF

                @pl.when(nxt < _GBLK)
                def _():
                    fire(nxt, s)
            return carry

        lax.fori_loop(0, _GBLK // _NBUF, body, 0)

    return k(table, idx)


def _sc_scatter_add(msg, dst, zeros_pad):
    nsteps = _EBLK // _NBUF_SC

    @functools.partial(
        pl.kernel,
        mesh=_sc_mesh(),
        compiler_params=_SC_PARAMS,
        out_type=jax.ShapeDtypeStruct((2 * _N_PAD, _H), jnp.float32),
        scratch_types=[
            pltpu.VMEM((_EBLK, _KE), jnp.int32),
            pltpu.VMEM((_NBUF_SC, _KE, _H), jnp.float32),
            pltpu.VMEM_SHARED((_N_PAD, _H), jnp.float32),
        ] + [pltpu.SemaphoreType.DMA] * (2 * _NBUF_SC),
    )
    def k(msg_h, dst_h, z_h, out_h, idx_v, bufs, agg, *sems):
        lsem = sems[:_NBUF_SC]
        csem = sems[_NBUF_SC:]
        cid = lax.axis_index("c")
        sid = lax.axis_index("s")
        wid = cid * _NS + sid
        pltpu.sync_copy(z_h.at[pl.ds(sid * _STRIPE, _STRIPE)],
                        agg.at[pl.ds(sid * _STRIPE, _STRIPE)])
        pltpu.sync_copy(dst_h.at[pl.ds(wid * _EBLK, _EBLK)], idx_v)
        plsc.subcore_barrier()

        def fire(blk, s):
            pltpu.async_copy(
                msg_h.at[pl.ds((wid * _EBLK + blk) * _KE, _KE)],
                bufs.at[s], lsem[s])

        for s in range(_NBUF_SC):
            fire(s, s)

        def body(kk, carry):
            for s in range(_NBUF_SC):
                blk = kk * _NBUF_SC + s
                pltpu.make_async_copy(
                    msg_h.at[pl.ds(0, _KE)], bufs.at[s], lsem[s]).wait()
                pltpu.async_copy(bufs.at[s], agg.at[idx_v.at[blk]],
                                 csem[s], add=True)
                pltpu.make_async_copy(
                    bufs.at[s], agg.at[idx_v.at[0]], csem[s]).wait()
                nxt = blk + _NBUF_SC

                @pl.when(nxt < _EBLK)
                def _():
                    fire(nxt, s)
            return carry

        lax.fori_loop(0, nsteps, body, 0)
        plsc.subcore_barrier()
        pltpu.sync_copy(agg.at[pl.ds(sid * _STRIPE, _STRIPE)],
                        out_h.at[pl.ds(cid * _N_PAD + sid * _STRIPE, _STRIPE)])

    return k(msg, dst, zeros_pad)


def _tc_node_linear(x, w):
    def body(x_ref, w_ref, o_ref):
        o_ref[...] = jnp.dot(x_ref[...], w_ref[...],
                             preferred_element_type=jnp.float32)

    return pl.pallas_call(
        body,
        grid=(_N // _TN,),
        in_specs=[pl.BlockSpec((_TN, _H), lambda i: (i, 0)),
                  pl.BlockSpec((_H, _H), lambda i: (0, 0))],
        out_specs=pl.BlockSpec((_TN, _H), lambda i: (i, 0)),
        out_shape=jax.ShapeDtypeStruct((_N, _H), jnp.float32),
    )(x, w)


def _tc_filter_msg(d2, ys, w1, b1, w2, b2):
    step = _CUTOFF / (_NRBF - 1)

    def body(d2_ref, ys_ref, w1_ref, b1_ref, w2_ref, b2_ref, o_ref):
        offs = lax.broadcasted_iota(jnp.int32, (1, _NRBF), 1).astype(
            jnp.float32) * step
        d = jnp.sqrt(d2_ref[...] + 1e-12)
        rbf = jnp.exp(_COEFF * (d - offs) ** 2)
        t = jnp.tanh(jnp.dot(rbf, w1_ref[...],
                             preferred_element_type=jnp.float32) + b1_ref[...])
        wf = jnp.dot(t, w2_ref[...],
                     preferred_element_type=jnp.float32) + b2_ref[...]
        cc = 0.5 * (jnp.cos(d * (math.pi / _CUTOFF)) + 1.0)
        cc = cc * (d < _CUTOFF).astype(jnp.float32)
        o_ref[...] = wf * cc * ys_ref[...]

    return pl.pallas_call(
        body,
        grid=(_E_PAD // _TE,),
        in_specs=[pl.BlockSpec((_TE, 1), lambda i: (i, 0)),
                  pl.BlockSpec((_TE, _H), lambda i: (i, 0)),
                  pl.BlockSpec((_NRBF, _H), lambda i: (0, 0)),
                  pl.BlockSpec((1, _H), lambda i: (0, 0)),
                  pl.BlockSpec((_H, _H), lambda i: (0, 0)),
                  pl.BlockSpec((1, _H), lambda i: (0, 0))],
        out_specs=pl.BlockSpec((_TE, _H), lambda i: (i, 0)),
        out_shape=jax.ShapeDtypeStruct((_E_PAD, _H), jnp.float32),
    )(d2, ys, w1, b1, w2, b2)


def _tc_update(agg_a, agg_b, x, w_out, b_out, w_lin, b_lin):
    def body(a_ref, b_ref, x_ref, wo_ref, bo_ref, wl_ref, bl_ref, o_ref):
        agg = a_ref[...] + b_ref[...]
        h = jnp.tanh(jnp.dot(agg, wo_ref[...],
                             preferred_element_type=jnp.float32) + bo_ref[...])
        h = jnp.dot(h, wl_ref[...],
                    preferred_element_type=jnp.float32) + bl_ref[...]
        o_ref[...] = x_ref[...] + h

    return pl.pallas_call(
        body,
        grid=(_N // _TN,),
        in_specs=[pl.BlockSpec((_TN, _H), lambda i: (i, 0)),
                  pl.BlockSpec((_TN, _H), lambda i: (i, 0)),
                  pl.BlockSpec((_TN, _H), lambda i: (i, 0)),
                  pl.BlockSpec((_H, _H), lambda i: (0, 0)),
                  pl.BlockSpec((1, _H), lambda i: (0, 0)),
                  pl.BlockSpec((_H, _H), lambda i: (0, 0)),
                  pl.BlockSpec((1, _H), lambda i: (0, 0))],
        out_specs=pl.BlockSpec((_TN, _H), lambda i: (i, 0)),
        out_shape=jax.ShapeDtypeStruct((_N, _H), jnp.float32),
    )(agg_a, agg_b, x, w_out, b_out, w_lin, b_lin)


def _tc_update_ylin(agg_a, agg_b, x, w_out, b_out, w_lin, b_lin, w_next):
    def body(a_ref, b_ref, x_ref, wo_ref, bo_ref, wl_ref, bl_ref, wn_ref,
             o_ref, y_ref):
        agg = a_ref[...] + b_ref[...]
        h = jnp.tanh(jnp.dot(agg, wo_ref[...],
                             preferred_element_type=jnp.float32) + bo_ref[...])
        h = jnp.dot(h, wl_ref[...],
                    preferred_element_type=jnp.float32) + bl_ref[...]
        xn = x_ref[...] + h
        o_ref[...] = xn
        y_ref[...] = jnp.dot(xn, wn_ref[...],
                             preferred_element_type=jnp.float32)

    return pl.pallas_call(
        body,
        grid=(_N // _TN,),
        in_specs=[pl.BlockSpec((_TN, _H), lambda i: (i, 0)),
                  pl.BlockSpec((_TN, _H), lambda i: (i, 0)),
                  pl.BlockSpec((_TN, _H), lambda i: (i, 0)),
                  pl.BlockSpec((_H, _H), lambda i: (0, 0)),
                  pl.BlockSpec((1, _H), lambda i: (0, 0)),
                  pl.BlockSpec((_H, _H), lambda i: (0, 0)),
                  pl.BlockSpec((1, _H), lambda i: (0, 0)),
                  pl.BlockSpec((_H, _H), lambda i: (0, 0))],
        out_specs=[pl.BlockSpec((_TN, _H), lambda i: (i, 0)),
                   pl.BlockSpec((_TN, _H), lambda i: (i, 0))],
        out_shape=[jax.ShapeDtypeStruct((_N, _H), jnp.float32),
                   jax.ShapeDtypeStruct((_N, _H), jnp.float32)],
    )(agg_a, agg_b, x, w_out, b_out, w_lin, b_lin, w_next)


def _tc_readout(x, seg, w1, b1, w2, b2):
    def body(x_ref, s_ref, w1_ref, b1_ref, w2_ref, b2_ref, o_ref):
        t = jnp.tanh(jnp.dot(x_ref[...], w1_ref[...],
                             preferred_element_type=jnp.float32) + b1_ref[...])
        e = jnp.dot(t, w2_ref[...],
                    preferred_element_type=jnp.float32) + b2_ref[...]
        cols = lax.broadcasted_iota(jnp.int32, (1, _NB), 1)
        m = (s_ref[...] == cols).astype(jnp.float32)
        part = jnp.sum(e * m, axis=0, keepdims=True)

        @pl.when(pl.program_id(0) == 0)
        def _():
            o_ref[...] = jnp.zeros_like(o_ref)

        o_ref[...] += part

    return pl.pallas_call(
        body,
        grid=(_N // _TN,),
        in_specs=[pl.BlockSpec((_TN, _H), lambda i: (i, 0)),
                  pl.BlockSpec((_TN, 1), lambda i: (i, 0)),
                  pl.BlockSpec((_H, _H // 2), lambda i: (0, 0)),
                  pl.BlockSpec((1, _H // 2), lambda i: (0, 0)),
                  pl.BlockSpec((_H // 2, 1), lambda i: (0, 0)),
                  pl.BlockSpec((1, 1), lambda i: (0, 0))],
        out_specs=pl.BlockSpec((1, _NB), lambda i: (0, 0)),
        out_shape=jax.ShapeDtypeStruct((1, _NB), jnp.float32),
    )(x, seg, w1, b1, w2, b2)


def kernel(pos, emb, filt_W1, filt_b1, filt_W2, filt_b2, lin_in_W, lin_out_W,
           lin_out_b, lin_W, lin_b, out1_W, out1_b, out2_W, out2_b,
           atom_types, edge_index, batch):
    f32 = jnp.float32
    pos = pos.astype(f32)
    src = edge_index[0].astype(jnp.int32)
    dst = edge_index[1].astype(jnp.int32)
    src_p = jnp.pad(src, (0, _E_PAD - _E))
    dst_p = jnp.pad(dst, (0, _E_PAD - _E), constant_values=_N)
    dst_blk = dst_p.reshape(_NW * _EBLK, _KE)
    posx = jnp.pad(pos[:, 0], (0, _N_PAD - _N))
    posy = jnp.pad(pos[:, 1], (0, _N_PAD - _N))
    posz = jnp.pad(pos[:, 2], (0, _N_PAD - _N))
    types_p = jnp.pad(atom_types.astype(jnp.int32), (0, _N_PAD - _N))
    zeros_pad = jnp.zeros((_N_PAD, _H), f32)
    seg = batch.astype(jnp.int32).reshape(_N, 1)

    d2 = _sc_edge_d2(posx, posy, posz, src_p, dst_p).reshape(_E_PAD, 1)
    x = _sc_gather_rows(emb.astype(f32), types_p, _NBLK, _KN)[:_N]
    y = _tc_node_linear(x, lin_in_W[0].astype(f32))

    for i in range(3):
        ys = _sc_gather_edges(y, src_p)
        msg = _tc_filter_msg(d2, ys,
                             filt_W1[i].astype(f32),
                             filt_b1[i].astype(f32).reshape(1, _H),
                             filt_W2[i].astype(f32),
                             filt_b2[i].astype(f32).reshape(1, _H))
        agg2 = _sc_scatter_add(msg, dst_blk, zeros_pad)
        args = (agg2[:_N], agg2[_N_PAD:_N_PAD + _N], x,
                lin_out_W[i].astype(f32),
                lin_out_b[i].astype(f32).reshape(1, _H),
                lin_W[i].astype(f32),
                lin_b[i].astype(f32).reshape(1, _H))
        if i < 2:
            x, y = _tc_update_ylin(*args, lin_in_W[i + 1].astype(f32))
        else:
            x = _tc_update(*args)

    energy = _tc_readout(x, seg,
                         out1_W.astype(f32),
                         out1_b.astype(f32).reshape(1, _H // 2),
                         out2_W.astype(f32),
                         out2_b.astype(f32).reshape(1, 1))
    return energy.reshape(_NB)


# speedup vs baseline: 1.2502x; 1.0033x over previous
import functools
import math

import jax
import jax.numpy as jnp
import numpy as np
from jax import lax
from jax.experimental import pallas as pl
from jax.experimental.pallas import tpu as pltpu
from jax.experimental.pallas import tpu_sc as plsc

_N = 10000
_E = 320000
_H = 128
_NRBF = 64
_NB = 64
_CUTOFF = 5.0

_NC = 2
_NS = 16
_NW = _NC * _NS

_E_PAD = 327680
_EW = _E_PAD // _NW
_KE = 128
_EBLK = _EW // _KE

_N_PAD = 10240
_KN = 40
_NBLK = (_N_PAD // _NW) // _KN

_STRIPE = _N_PAD // _NS

_TN = 2000
_TE = 1024

_OFFS = np.linspace(0.0, _CUTOFF, _NRBF).astype(np.float32)
_COEFF = float(-0.5 / (_OFFS[1] - _OFFS[0]) ** 2)

_NBUF = 4
_NBUF_SC = 2


def _sc_mesh():
    return plsc.VectorSubcoreMesh(core_axis_name="c", subcore_axis_name="s")


_SC_PARAMS = pltpu.CompilerParams(needs_layout_passes=False)


def _sc_edge_d2(posx, posy, posz, src, dst):
    @functools.partial(
        pl.kernel,
        mesh=_sc_mesh(),
        compiler_params=_SC_PARAMS,
        out_type=jax.ShapeDtypeStruct((_E_PAD,), jnp.float32),
        scratch_types=[
            pltpu.VMEM((_N_PAD,), jnp.float32),
            pltpu.VMEM((_N_PAD,), jnp.float32),
            pltpu.VMEM((_N_PAD,), jnp.float32),
            pltpu.VMEM((_EW,), jnp.int32),
            pltpu.VMEM((_EW,), jnp.int32),
            pltpu.VMEM((_EW,), jnp.float32),
        ],
    )
    def k(px_h, py_h, pz_h, src_h, dst_h, out_h, px, py, pz, sb, db, d2b):
        cid = lax.axis_index("c")
        sid = lax.axis_index("s")
        base = (cid * _NS + sid) * _EW
        pltpu.sync_copy(px_h, px)
        pltpu.sync_copy(py_h, py)
        pltpu.sync_copy(pz_h, pz)
        pltpu.sync_copy(src_h.at[pl.ds(base, _EW)], sb)
        pltpu.sync_copy(dst_h.at[pl.ds(base, _EW)], db)

        def body(j, carry):
            o = j * 16
            si = sb[pl.ds(o, 16)]
            di = db[pl.ds(o, 16)]
            dx = plsc.load_gather(px, [di]) - plsc.load_gather(px, [si])
            dy = plsc.load_gather(py, [di]) - plsc.load_gather(py, [si])
            dz = plsc.load_gather(pz, [di]) - plsc.load_gather(pz, [si])
            d2b[pl.ds(o, 16)] = dx * dx + dy * dy + dz * dz
            return carry

        lax.fori_loop(0, _EW // 16, body, 0)
        pltpu.sync_copy(d2b, out_h.at[pl.ds(base, _EW)])

    return k(posx, posy, posz, src, dst)


def _sc_gather_rows(table, idx, nblocks, k_rows):
    total = _NW * nblocks * k_rows
    per_w = nblocks * k_rows
    nsteps = nblocks // _NBUF

    @functools.partial(
        pl.kernel,
        mesh=_sc_mesh(),
        compiler_params=_SC_PARAMS,
        out_type=jax.ShapeDtypeStruct((total, _H), jnp.float32),
        scratch_types=[
            pltpu.VMEM((per_w,), jnp.int32),
            pltpu.VMEM((_NBUF, k_rows, _H), jnp.float32),
        ] + [pltpu.SemaphoreType.DMA] * (2 * _NBUF),
    )
    def k(table_h, idx_h, out_h, idx_v, bufs, *sems):
        gsem = sems[:_NBUF]
        ssem = sems[_NBUF:]
        cid = lax.axis_index("c")
        sid = lax.axis_index("s")
        wid = cid * _NS + sid
        base = wid * per_w
        pltpu.sync_copy(idx_h.at[pl.ds(base, per_w)], idx_v)

        def fire(blk, s):
            pltpu.async_copy(
                table_h.at[idx_v.at[pl.ds(blk * k_rows, k_rows)]],
                bufs.at[s], gsem[s])

        for s in range(_NBUF):
            fire(s, s)

        def body(kk, carry):
            for s in range(_NBUF):
                blk = kk * _NBUF + s
                pltpu.make_async_copy(
                    table_h.at[idx_v.at[pl.ds(0, k_rows)]],
                    bufs.at[s], gsem[s]).wait()
                pltpu.async_copy(
                    bufs.at[s],
                    out_h.at[pl.ds(base + blk * k_rows, k_rows)], ssem[s])
                pltpu.make_async_copy(
                    bufs.at[s],
                    out_h.at[pl.ds(base, k_rows)], ssem[s]).wait()
                nxt = blk + _NBUF

                @pl.when(nxt < nblocks)
                def _():
                    fire(nxt, s)
            return carry

        lax.fori_loop(0, nsteps, body, 0)

    return k(table, idx)


_KG = 64
_GBLK = _EW // _KG
_YSTRIPE = _N // _NS


def _sc_gather_edges(table, idx):
    @functools.partial(
        pl.kernel,
        mesh=_sc_mesh(),
        compiler_params=_SC_PARAMS,
        out_type=jax.ShapeDtypeStruct((_E_PAD, _H), jnp.float32),
        scratch_types=[
            pltpu.VMEM((_EW,), jnp.int32),
            pltpu.VMEM((_NBUF, _KG, _H), jnp.float32),
            pltpu.VMEM_SHARED((_N, _H), jnp.float32),
        ] + [pltpu.SemaphoreType.DMA] * (2 * _NBUF),
    )
    def k(table_h, idx_h, out_h, idx_v, bufs, ytab, *sems):
        gsem = sems[:_NBUF]
        ssem = sems[_NBUF:]
        cid = lax.axis_index("c")
        sid = lax.axis_index("s")
        wid = cid * _NS + sid
        base = wid * _EW

        @pl.when(sid < 15)
        def _():
            pltpu.sync_copy(table_h.at[pl.ds(sid * 640, 640)],
                            ytab.at[pl.ds(sid * 640, 640)])

        @pl.when(sid == 15)
        def _():
            pltpu.sync_copy(table_h.at[pl.ds(9600, _N - 9600)],
                            ytab.at[pl.ds(9600, _N - 9600)])

        pltpu.sync_copy(idx_h.at[pl.ds(base, _EW)], idx_v)
        plsc.subcore_barrier()

        def fire(blk, s):
            pltpu.async_copy(
                ytab.at[idx_v.at[pl.ds(blk * _KG, _KG)]],
                bufs.at[s], gsem[s])

        for s in range(_NBUF):
            fire(s, s)

        def body(kk, carry):
            for s in range(_NBUF):
                blk = kk * _NBUF + s
                pltpu.make_async_copy(
                    table_h.at[pl.ds(0, _KG)],
                    bufs.at[s], gsem[s]).wait()
                pltpu.async_copy(
                    bufs.at[s],
                    out_h.at[pl.ds(base + blk * _KG, _KG)], ssem[s])
                pltpu.make_async_copy(
                    bufs.at[s],
                    out_h.at[pl.ds(0, _KG)], ssem[s]).wait()
                nxt = blk + _NBUF

                @pl.when(nxt < _GBLK)
                def _():
                    fire(nxt, s)
            return carry

        lax.fori_loop(0, _GBLK // _NBUF, body, 0)

    return k(table, idx)


def _sc_scatter_add(msg, dst, zeros_pad):
    nsteps = _EBLK // _NBUF_SC

    @functools.partial(
        pl.kernel,
        mesh=_sc_mesh(),
        compiler_params=_SC_PARAMS,
        out_type=jax.ShapeDtypeStruct((2 * _N_PAD, _H), jnp.float32),
        scratch_types=[
            pltpu.VMEM((_EBLK, _KE), jnp.int32),
            pltpu.VMEM((_NBUF_SC, _KE, _H), jnp.float32),
            pltpu.VMEM_SHARED((_N_PAD, _H), jnp.float32),
        ] + [pltpu.SemaphoreType.DMA] * (2 * _NBUF_SC),
    )
    def k(msg_h, dst_h, z_h, out_h, idx_v, bufs, agg, *sems):
        lsem = sems[:_NBUF_SC]
        csem = sems[_NBUF_SC:]
        cid = lax.axis_index("c")
        sid = lax.axis_index("s")
        wid = cid * _NS + sid
        pltpu.sync_copy(z_h.at[pl.ds(sid * _STRIPE, _STRIPE)],
                        agg.at[pl.ds(sid * _STRIPE, _STRIPE)])
        pltpu.sync_copy(dst_h.at[pl.ds(wid * _EBLK, _EBLK)], idx_v)
        plsc.subcore_barrier()

        def fire(blk, s):
            pltpu.async_copy(
                msg_h.at[pl.ds((wid * _EBLK + blk) * _KE, _KE)],
                bufs.at[s], lsem[s])

        for s in range(_NBUF_SC):
            fire(s, s)

        def body(kk, carry):
            for s in range(_NBUF_SC):
                blk = kk * _NBUF_SC + s
                pltpu.make_async_copy(
                    msg_h.at[pl.ds(0, _KE)], bufs.at[s], lsem[s]).wait()
                pltpu.async_copy(bufs.at[s], agg.at[idx_v.at[blk]],
                                 csem[s], add=True)
                pltpu.make_async_copy(
                    bufs.at[s], agg.at[idx_v.at[0]], csem[s]).wait()
                nxt = blk + _NBUF_SC

                @pl.when(nxt < _EBLK)
                def _():
                    fire(nxt, s)
            return carry

        lax.fori_loop(0, nsteps, body, 0)
        plsc.subcore_barrier()
        pltpu.sync_copy(agg.at[pl.ds(sid * _STRIPE, _STRIPE)],
                        out_h.at[pl.ds(cid * _N_PAD + sid * _STRIPE, _STRIPE)])

    return k(msg, dst, zeros_pad)


def _tc_node_linear(x, w):
    def body(x_ref, w_ref, o_ref):
        o_ref[...] = jnp.dot(x_ref[...], w_ref[...],
                             preferred_element_type=jnp.float32)

    return pl.pallas_call(
        body,
        grid=(_N // _TN,),
        in_specs=[pl.BlockSpec((_TN, _H), lambda i: (i, 0)),
                  pl.BlockSpec((_H, _H), lambda i: (0, 0))],
        out_specs=pl.BlockSpec((_TN, _H), lambda i: (i, 0)),
        out_shape=jax.ShapeDtypeStruct((_N, _H), jnp.float32),
    )(x, w)


def _tc_filter_msg(d2, ys, w1, b1, w2, b2):
    step = _CUTOFF / (_NRBF - 1)

    def body(d2_ref, ys_ref, w1_ref, b1_ref, w2_ref, b2_ref, o_ref):
        offs = lax.broadcasted_iota(jnp.int32, (1, _NRBF), 1).astype(
            jnp.float32) * step
        d = jnp.sqrt(d2_ref[...] + 1e-12)
        rbf = jnp.exp(_COEFF * (d - offs) ** 2)
        t = jnp.tanh(jnp.dot(rbf, w1_ref[...],
                             preferred_element_type=jnp.float32) + b1_ref[...])
        wf = jnp.dot(t, w2_ref[...],
                     preferred_element_type=jnp.float32) + b2_ref[...]
        cc = 0.5 * (jnp.cos(d * (math.pi / _CUTOFF)) + 1.0)
        cc = cc * (d < _CUTOFF).astype(jnp.float32)
        o_ref[...] = wf * cc * ys_ref[...]

    return pl.pallas_call(
        body,
        grid=(_E_PAD // _TE,),
        in_specs=[pl.BlockSpec((_TE, 1), lambda i: (i, 0)),
                  pl.BlockSpec((_TE, _H), lambda i: (i, 0)),
                  pl.BlockSpec((_NRBF, _H), lambda i: (0, 0)),
                  pl.BlockSpec((1, _H), lambda i: (0, 0)),
                  pl.BlockSpec((_H, _H), lambda i: (0, 0)),
                  pl.BlockSpec((1, _H), lambda i: (0, 0))],
        out_specs=pl.BlockSpec((_TE, _H), lambda i: (i, 0)),
        out_shape=jax.ShapeDtypeStruct((_E_PAD, _H), jnp.float32),
    )(d2, ys, w1, b1, w2, b2)


def _tc_update(agg_a, agg_b, x, w_out, b_out, w_lin, b_lin):
    def body(a_ref, b_ref, x_ref, wo_ref, bo_ref, wl_ref, bl_ref, o_ref):
        agg = a_ref[...] + b_ref[...]
        h = jnp.tanh(jnp.dot(agg, wo_ref[...],
                             preferred_element_type=jnp.float32) + bo_ref[...])
        h = jnp.dot(h, wl_ref[...],
                    preferred_element_type=jnp.float32) + bl_ref[...]
        o_ref[...] = x_ref[...] + h

    return pl.pallas_call(
        body,
        grid=(_N // _TN,),
        in_specs=[pl.BlockSpec((_TN, _H), lambda i: (i, 0)),
                  pl.BlockSpec((_TN, _H), lambda i: (i, 0)),
                  pl.BlockSpec((_TN, _H), lambda i: (i, 0)),
                  pl.BlockSpec((_H, _H), lambda i: (0, 0)),
                  pl.BlockSpec((1, _H), lambda i: (0, 0)),
                  pl.BlockSpec((_H, _H), lambda i: (0, 0)),
                  pl.BlockSpec((1, _H), lambda i: (0, 0))],
        out_specs=pl.BlockSpec((_TN, _H), lambda i: (i, 0)),
        out_shape=jax.ShapeDtypeStruct((_N, _H), jnp.float32),
    )(agg_a, agg_b, x, w_out, b_out, w_lin, b_lin)


def _tc_update_ylin(agg_a, agg_b, x, w_out, b_out, w_lin, b_lin, w_next):
    def body(a_ref, b_ref, x_ref, wo_ref, bo_ref, wl_ref, bl_ref, wn_ref,
             o_ref, y_ref):
        agg = a_ref[...] + b_ref[...]
        h = jnp.tanh(jnp.dot(agg, wo_ref[...],
                             preferred_element_type=jnp.float32) + bo_ref[...])
        h = jnp.dot(h, wl_ref[...],
                    preferred_element_type=jnp.float32) + bl_ref[...]
        xn = x_ref[...] + h
        o_ref[...] = xn
        y_ref[...] = jnp.dot(xn, wn_ref[...],
                             preferred_element_type=jnp.float32)

    return pl.pallas_call(
        body,
        grid=(_N // _TN,),
        in_specs=[pl.BlockSpec((_TN, _H), lambda i: (i, 0)),
                  pl.BlockSpec((_TN, _H), lambda i: (i, 0)),
                  pl.BlockSpec((_TN, _H), lambda i: (i, 0)),
                  pl.BlockSpec((_H, _H), lambda i: (0, 0)),
                  pl.BlockSpec((1, _H), lambda i: (0, 0)),
                  pl.BlockSpec((_H, _H), lambda i: (0, 0)),
                  pl.BlockSpec((1, _H), lambda i: (0, 0)),
                  pl.BlockSpec((_H, _H), lambda i: (0, 0))],
        out_specs=[pl.BlockSpec((_TN, _H), lambda i: (i, 0)),
                   pl.BlockSpec((_TN, _H), lambda i: (i, 0))],
        out_shape=[jax.ShapeDtypeStruct((_N, _H), jnp.float32),
                   jax.ShapeDtypeStruct((_N, _H), jnp.float32)],
    )(agg_a, agg_b, x, w_out, b_out, w_lin, b_lin, w_next)


def _tc_readout(x, seg, w1, b1, w2, b2):
    def body(x_ref, s_ref, w1_ref, b1_ref, w2_ref, b2_ref, o_ref):
        t = jnp.tanh(jnp.dot(x_ref[...], w1_ref[...],
                             preferred_element_type=jnp.float32) + b1_ref[...])
        e = jnp.dot(t, w2_ref[...],
                    preferred_element_type=jnp.float32) + b2_ref[...]
        cols = lax.broadcasted_iota(jnp.int32, (1, _NB), 1)
        m = (s_ref[...] == cols).astype(jnp.float32)
        part = jnp.sum(e * m, axis=0, keepdims=True)

        @pl.when(pl.program_id(0) == 0)
        def _():
            o_ref[...] = jnp.zeros_like(o_ref)

        o_ref[...] += part

    return pl.pallas_call(
        body,
        grid=(_N // _TN,),
        in_specs=[pl.BlockSpec((_TN, _H), lambda i: (i, 0)),
                  pl.BlockSpec((_TN, 1), lambda i: (i, 0)),
                  pl.BlockSpec((_H, _H // 2), lambda i: (0, 0)),
                  pl.BlockSpec((1, _H // 2), lambda i: (0, 0)),
                  pl.BlockSpec((_H // 2, 1), lambda i: (0, 0)),
                  pl.BlockSpec((1, 1), lambda i: (0, 0))],
        out_specs=pl.BlockSpec((1, _NB), lambda i: (0, 0)),
        out_shape=jax.ShapeDtypeStruct((1, _NB), jnp.float32),
    )(x, seg, w1, b1, w2, b2)


def kernel(pos, emb, filt_W1, filt_b1, filt_W2, filt_b2, lin_in_W, lin_out_W,
           lin_out_b, lin_W, lin_b, out1_W, out1_b, out2_W, out2_b,
           atom_types, edge_index, batch):
    f32 = jnp.float32
    pos = pos.astype(f32)
    src = edge_index[0].astype(jnp.int32)
    dst = edge_index[1].astype(jnp.int32)
    src_p = jnp.pad(src, (0, _E_PAD - _E))
    dst_p = jnp.pad(dst, (0, _E_PAD - _E), constant_values=_N)
    dst_blk = dst_p.reshape(_NW * _EBLK, _KE)
    posx = jnp.pad(pos[:, 0], (0, _N_PAD - _N))
    posy = jnp.pad(pos[:, 1], (0, _N_PAD - _N))
    posz = jnp.pad(pos[:, 2], (0, _N_PAD - _N))
    types_p = jnp.pad(atom_types.astype(jnp.int32), (0, _N_PAD - _N))
    zeros_pad = jnp.zeros((_N_PAD, _H), f32)
    seg = batch.astype(jnp.int32).reshape(_N, 1)

    d2 = _sc_edge_d2(posx, posy, posz, src_p, dst_p).reshape(_E_PAD, 1)
    x = _sc_gather_rows(emb.astype(f32), types_p, _NBLK, _KN)[:_N]
    y = _tc_node_linear(x, lin_in_W[0].astype(f32))

    for i in range(3):
        ys = _sc_gather_edges(y, src_p)
        msg = _tc_filter_msg(d2, ys,
                             filt_W1[i].astype(f32),
                             filt_b1[i].astype(f32).reshape(1, _H),
                             filt_W2[i].astype(f32),
                             filt_b2[i].astype(f32).reshape(1, _H))
        agg2 = _sc_scatter_add(msg, dst_blk, zeros_pad)
        args = (agg2[:_N], agg2[_N_PAD:_N_PAD + _N], x,
                lin_out_W[i].astype(f32),
                lin_out_b[i].astype(f32).reshape(1, _H),
                lin_W[i].astype(f32),
                lin_b[i].astype(f32).reshape(1, _H))
        if i < 2:
            x, y = _tc_update_ylin(*args, lin_in_W[i + 1].astype(f32))
        else:
            x = _tc_update(*args)

    energy = _tc_readout(x, seg,
                         out1_W.astype(f32),
                         out1_b.astype(f32).reshape(1, _H // 2),
                         out2_W.astype(f32),
                         out2_b.astype(f32).reshape(1, 1))
    return energy.reshape(_NB)
